# baseline (speedup 1.0000x reference)
import jax
import jax.numpy as jnp
from jax import lax
from jax.experimental import pallas as pl
from jax.experimental.pallas import tpu as pltpu

_ROWS = 16
_COLS = 4096
_BLK = 128
_NBLK = _COLS // _BLK


def _body(x_ref, o_ref):
    x = x_ref[...]
    i = lax.broadcasted_iota(jnp.int32, (_BLK, _BLK), 0)
    j = lax.broadcasted_iota(jnp.int32, (_BLK, _BLK), 1)
    tri = (i <= j).astype(jnp.int8)
    xcat = jnp.concatenate(
        [lax.slice(x, (0, b * _BLK), (_ROWS, (b + 1) * _BLK))
         for b in range(_NBLK)], axis=0)
    call = lax.dot(xcat, tri, preferred_element_type=jnp.int32)
    cbs = []
    incl = []
    for b in range(_NBLK):
        cb = lax.slice(call, (b * _ROWS, 0), ((b + 1) * _ROWS, _BLK))
        cbs.append(cb)
        incl.append(lax.slice(cb, (0, _BLK - 1), (_ROWS, _BLK)))
    d = 1
    while d < _NBLK:
        incl = [incl[b] if b < d else incl[b] + incl[b - d]
                for b in range(_NBLK)]
        d *= 2
    for b in range(_NBLK):
        ob = cbs[b] if b == 0 else cbs[b] + incl[b - 1]
        o_ref[:, b * _BLK:(b + 1) * _BLK] = ob


@jax.jit
def kernel(masks):
    x8 = masks.view(jnp.int8)
    return pl.pallas_call(
        _body,
        out_shape=jax.ShapeDtypeStruct((_ROWS, _COLS), jnp.int32),
        compiler_params=pltpu.CompilerParams(allow_input_fusion=[True]),
    )(x8)
